# baseline (speedup 1.0000x reference)
import functools

import jax
import jax.numpy as jnp
from jax import lax
from jax.experimental import pallas as pl
from jax.experimental.pallas import tpu as pltpu
from jax.experimental.pallas import tpu_sc as plsc

_N = 10000
_NPAD = 10240
_E = 320000
_R = 256
_NT = _NPAD // _R
_K = 128
_NSUB = 16
_NCORE = 2
_H = 256


_EPAD = 327680
_EPS = _EPAD // _NSUB
_NCH = _EPS // _K
_NBUF = 4


_KB = 1024
_NBLK = _EPS // _KB
_CPB = _KB // _K


def _seg_sum_body(dh, xsplit, srcr, dstr, zeros, out,
                  src_a, src_b, dst_a, dst_b, dv_a, dv_b, rows_a, rows_b, acc,
                  isem_a, isem_b, gsem_a, gsem_b, ssem_a, ssem_b):
    srcblk = (src_a, src_b)
    dstblk = (dst_a, dst_b)
    dstv = (dv_a, dv_b)
    rows = (rows_a, rows_b)
    isem = (isem_a, isem_b)
    gsem = (gsem_a, gsem_b)
    ssem = (ssem_a, ssem_b)
    cax = lax.axis_index("c")
    s = lax.axis_index("s")
    rps = _NPAD // _NSUB
    pltpu.sync_copy(zeros, acc.at[pl.ds(s * rps, rps)])
    coff = cax * _NPAD

    def idx_copies(b):
        sb = b % 2
        return (
            pltpu.make_async_copy(srcr.at[pl.ds(s * _EPS + b * _KB, _KB)],
                                  srcblk[sb], isem[sb]),
            pltpu.make_async_copy(dstr.at[pl.ds(s * _EPS + b * _KB, _KB)],
                                  dstblk[sb], isem[sb]),
        )

    def fire_idx(b):
        for d in idx_copies(b):
            d.start()

    def wait_idx_and_fix(b):
        sb = b % 2
        for d in idx_copies(b):
            d.wait()

        def fix(i, carry):
            srcblk[sb][pl.ds(i * 16, 16)] = srcblk[sb][pl.ds(i * 16, 16)] + coff
            return carry

        lax.fori_loop(0, _KB // 16, fix, 0)

    def gdesc(t):
        b, cc = divmod(t, _CPB)
        j = t % 2
        return pltpu.make_async_copy(
            xsplit.at[srcblk[b % 2].at[pl.ds(cc * _K, _K)]], rows[j], gsem[j])

    def sdesc(t):
        j = t % 2
        return pltpu.make_async_copy(rows[j], acc.at[dstv[j]], ssem[j])

    fire_idx(0)
    wait_idx_and_fix(0)
    plsc.subcore_barrier()
    gdesc(0).start()

    for b in range(_NBLK):
        for cc in range(_CPB):
            t = b * _CPB + cc
            if t >= 1:
                sdesc(t - 1).wait()
            if cc == 2 and b + 1 < _NBLK:
                fire_idx(b + 1)
            if cc == 6 and b + 1 < _NBLK:
                wait_idx_and_fix(b + 1)
            for g in range(_K // 16):
                dstv[t % 2][pl.ds(g * 16, 16)] = (
                    dstblk[b % 2][pl.ds(cc * _K + g * 16, 16)])
            gdesc(t).wait()
            if t + 1 < _NCH:
                gdesc(t + 1).start()
            sdesc(t).start(add=True)
    sdesc(_NCH - 1).wait()

    plsc.subcore_barrier()
    pltpu.sync_copy(acc.at[pl.ds(s * rps, rps)],
                    out.at[pl.ds(coff + s * rps, rps)])


def _seg_sum(xsplit, srcr, dstr, dh):
    zeros = jnp.zeros((_NPAD // _NSUB, dh), jnp.float32)
    mesh = plsc.VectorSubcoreMesh(core_axis_name="c", subcore_axis_name="s",
                                  num_cores=_NCORE, num_subcores=_NSUB)
    f = pl.kernel(
        functools.partial(_seg_sum_body, dh),
        out_type=jax.ShapeDtypeStruct((_NCORE * _NPAD, dh), jnp.float32),
        mesh=mesh,
        scratch_types=(
            [pltpu.VMEM((_KB,), jnp.int32) for _ in range(4)]
            + [pltpu.VMEM((_K,), jnp.int32) for _ in range(2)]
            + [pltpu.VMEM((_K, dh), jnp.float32) for _ in range(2)]
            + [pltpu.VMEM_SHARED((_NPAD, dh), jnp.float32)]
            + [pltpu.SemaphoreType.DMA for _ in range(6)]
        ),
    )
    return f(xsplit, srcr, dstr, zeros)


def _bn_lrelu_split(z, g, bb, m, v, o_ref):
    scale = g * lax.rsqrt(v + 1e-5)
    z = (z - m) * scale + bb
    z = jnp.where(z >= 0, z, 0.01 * z)
    o_ref[0] = z[:, :128]
    o_ref[1] = z[:, 128:]


def _l1_body(x_ref, alo_ref, ahi_ref, ws_ref, wn_ref, b_ref, g_ref, bb_ref,
             m_ref, v_ref, o_ref):
    x = x_ref[...]
    a = jnp.concatenate([alo_ref[:, :64], ahi_ref[:, :64]], axis=1)
    deg = alo_ref[:, 64:65]
    hn = a * (1.0 / jnp.maximum(deg, 1.0))
    z = (jnp.dot(x, ws_ref[...], preferred_element_type=jnp.float32)
         + jnp.dot(hn, wn_ref[...], preferred_element_type=jnp.float32)
         + b_ref[...])
    _bn_lrelu_split(z, g_ref[...], bb_ref[...], m_ref[...], v_ref[...], o_ref)


def _l23_body(xlo_ref, xhi_ref, alo_ref, ahi_ref, d_ref, ws_ref, wn_ref,
              b_ref, g_ref, bb_ref, m_ref, v_ref, o_ref):
    x = jnp.concatenate([xlo_ref[...], xhi_ref[...]], axis=1)
    a = jnp.concatenate([alo_ref[...], ahi_ref[...]], axis=1)
    deg = d_ref[:, 64:65]
    hn = a * (1.0 / jnp.maximum(deg, 1.0))
    z = (jnp.dot(x, ws_ref[...], preferred_element_type=jnp.float32)
         + jnp.dot(hn, wn_ref[...], preferred_element_type=jnp.float32)
         + b_ref[...])
    _bn_lrelu_split(z, g_ref[...], bb_ref[...], m_ref[...], v_ref[...], o_ref)


def _full(shape):
    return pl.BlockSpec(shape, lambda i: (0,) * len(shape))


def _layer1(h_pad, agg1, Ws, Wn, b, g, bb, m, v):
    return pl.pallas_call(
        _l1_body,
        grid=(_NT,),
        in_specs=[
            pl.BlockSpec((_R, 128), lambda i: (i, 0)),
            pl.BlockSpec((_R, 128), lambda i: (i, 0)),
            pl.BlockSpec((_R, 128), lambda i: (_NT + i, 0)),
            _full((128, _H)), _full((128, _H)),
            _full((1, _H)), _full((1, _H)), _full((1, _H)),
            _full((1, _H)), _full((1, _H)),
        ],
        out_specs=pl.BlockSpec((2, _R, 128), lambda i: (0, i, 0)),
        out_shape=jax.ShapeDtypeStruct((2, _NPAD, 128), jnp.float32),
    )(h_pad, agg1, agg1, Ws, Wn, b, g, bb, m, v)


def _layer23(xsplit, agg, agg1, Ws, Wn, b, g, bb, m, v):
    return pl.pallas_call(
        _l23_body,
        grid=(_NT,),
        in_specs=[
            pl.BlockSpec((_R, 128), lambda i: (i, 0)),
            pl.BlockSpec((_R, 128), lambda i: (_NT + i, 0)),
            pl.BlockSpec((_R, 128), lambda i: (i, 0)),
            pl.BlockSpec((_R, 128), lambda i: (_NT + i, 0)),
            pl.BlockSpec((_R, 128), lambda i: (i, 0)),
            _full((_H, _H)), _full((_H, _H)),
            _full((1, _H)), _full((1, _H)), _full((1, _H)),
            _full((1, _H)), _full((1, _H)),
        ],
        out_specs=pl.BlockSpec((2, _R, 128), lambda i: (0, i, 0)),
        out_shape=jax.ShapeDtypeStruct((2, _NPAD, 128), jnp.float32),
    )(xsplit, xsplit, agg, agg, agg1, Ws, Wn, b, g, bb, m, v)


def _final_body(xlo_ref, xhi_ref, f1w_ref, f1b_ref, f2w_ref, f2b_ref,
                f3w_ref, f3b_ref, o_ref, acc_ref):
    i = pl.program_id(0)
    xt = jnp.concatenate([xlo_ref[...], xhi_ref[...]], axis=1)
    row = i * _R + lax.broadcasted_iota(jnp.int32, (_R, 1), 0)
    xt = jnp.where(row < _N, xt, 0.0)

    @pl.when(i == 0)
    def _():
        acc_ref[...] = jnp.zeros_like(acc_ref)

    acc_ref[...] += jnp.sum(xt, axis=0, keepdims=True)

    @pl.when(i == _NT - 1)
    def _():
        hg = acc_ref[...] * (1.0 / _N)
        y = hg @ f1w_ref[...] + f1b_ref[...]
        y = jnp.where(y >= 0, y, 0.01 * y)
        y = y @ f2w_ref[...] + f2b_ref[...]
        y = jnp.where(y >= 0, y, 0.01 * y)
        o_ref[...] = y @ f3w_ref[...] + f3b_ref[...]


def _final(xsplit, f1w, f1b, f2w, f2b, f3w, f3b):
    return pl.pallas_call(
        _final_body,
        grid=(_NT,),
        in_specs=[
            pl.BlockSpec((_R, 128), lambda i: (i, 0)),
            pl.BlockSpec((_R, 128), lambda i: (_NT + i, 0)),
            _full((_H, _H)), _full((1, _H)),
            _full((_H, 1024)), _full((1, 1024)),
            _full((1024, 128)), _full((1, 128)),
        ],
        out_specs=pl.BlockSpec((1, 128), lambda i: (0, 0)),
        out_shape=jax.ShapeDtypeStruct((1, 128), jnp.float32),
        scratch_shapes=[pltpu.VMEM((1, _H), jnp.float32)],
    )(xsplit, xsplit, f1w, f1b, f2w, f2b, f3w, f3b)


def kernel(h, edge_index, Ws1, Wn1, b1, Ws2, Wn2, b2, Ws3, Wn3, b3,
           bn1g, bn1b, bn1m, bn1v, bn2g, bn2b, bn2m, bn2v, bn3g, bn3b,
           bn3m, bn3v, fc1W, fc1b, fc2W, fc2b, fc3W, fc3b):
    f32 = jnp.float32
    npd = _EPAD - _E
    src = jnp.concatenate([edge_index[0], jnp.zeros((npd,), jnp.int32)])
    dst = jnp.concatenate([edge_index[1], jnp.full((npd,), _NPAD - 1, jnp.int32)])

    h_pad = jnp.zeros((_NPAD, 128), f32).at[:_N].set(h)
    onescol = jnp.ones((_NPAD, 1), f32)
    zpad = jnp.zeros((_NPAD, 63), f32)
    zpad2 = jnp.zeros((_NPAD, 64), f32)
    hsplit = jnp.concatenate([
        jnp.concatenate([h_pad[:, :64], onescol, zpad], axis=1),
        jnp.concatenate([h_pad[:, 64:], zpad2], axis=1)], axis=0)

    r1 = lambda a: a.reshape(1, -1)

    agg1 = _seg_sum(hsplit, src, dst, 128)
    x1 = _layer1(h_pad, agg1, Ws1, Wn1, r1(b1), r1(bn1g), r1(bn1b),
                 r1(bn1m), r1(bn1v)).reshape(_NCORE * _NPAD, 128)
    agg2 = _seg_sum(x1, src, dst, 128)
    x2 = _layer23(x1, agg2, agg1, Ws2, Wn2, r1(b2), r1(bn2g), r1(bn2b),
                  r1(bn2m), r1(bn2v)).reshape(_NCORE * _NPAD, 128)
    agg3 = _seg_sum(x2, src, dst, 128)
    x3 = _layer23(x2, agg3, agg1, Ws3, Wn3, r1(b3), r1(bn3g), r1(bn3b),
                  r1(bn3m), r1(bn3v)).reshape(_NCORE * _NPAD, 128)

    f3Wp = jnp.zeros((1024, 128), f32).at[:, :18].set(fc3W)
    f3bp = jnp.zeros((1, 128), f32).at[:, :18].set(r1(fc3b))
    y = _final(x3, fc1W, r1(fc1b), fc2W, r1(fc2b), f3Wp, f3bp)
    return y[:, :18]


# speedup vs baseline: 1.0032x; 1.0032x over previous
import functools

import jax
import jax.numpy as jnp
from jax import lax
from jax.experimental import pallas as pl
from jax.experimental.pallas import tpu as pltpu
from jax.experimental.pallas import tpu_sc as plsc

_N = 10000
_NPAD = 10240
_E = 320000
_R = 256
_NT = _NPAD // _R
_K = 128
_NSUB = 16
_NCORE = 2
_H = 256


_EPAD = 327680
_EPS = _EPAD // _NSUB
_NCH = _EPS // _K
_NBUF = 4


def _seg_sum_body(dh, xsplit, srcr, dstr, zeros, out,
                  sl_a, sl_b, ix_a, ix_b, dv0, dv1, dv2, dv3,
                  rows_a, rows_b, acc,
                  isem_a, isem_b, gsem_a, gsem_b, ssem_a, ssem_b):
    srcload = (sl_a, sl_b)
    idxv = (ix_a, ix_b)
    dstv = (dv0, dv1, dv2, dv3)
    rows = (rows_a, rows_b)
    isem = (isem_a, isem_b)
    gsem = (gsem_a, gsem_b)
    ssem = (ssem_a, ssem_b)
    cax = lax.axis_index("c")
    s = lax.axis_index("s")
    rps = _NPAD // _NSUB
    pltpu.sync_copy(zeros, acc.at[pl.ds(s * rps, rps)])
    coff = cax * _NPAD
    ebase = s * _EPS

    def idescs(t, j, q):
        off = ebase + t * _K
        return (
            pltpu.make_async_copy(srcr.at[pl.ds(off, _K)], srcload[j], isem[j]),
            pltpu.make_async_copy(dstr.at[pl.ds(off, _K)], dstv[q], isem[j]),
        )

    def gdesc(j):
        return pltpu.make_async_copy(xsplit.at[idxv[j]], rows[j], gsem[j])

    def sdesc(j, q):
        return pltpu.make_async_copy(rows[j], acc.at[dstv[q]], ssem[j])

    def step(t, j, q, wait_sc, fire_idx):
        if wait_sc:
            sdesc(j, q).wait()
        for d in idescs(t, j, q):
            d.wait()
        for g in range(_K // 16):
            idxv[j][pl.ds(g * 16, 16)] = srcload[j][pl.ds(g * 16, 16)] + coff
        if fire_idx:
            for d in idescs(t + 2, j, (q + 2) % 4):
                d.start()
        gdesc(j).start()
        gdesc(j).wait()
        sdesc(j, q).start(add=True)

    for d in idescs(0, 0, 0):
        d.start()
    for d in idescs(1, 1, 1):
        d.start()
    plsc.subcore_barrier()

    step(0, 0, 0, False, True)
    step(1, 1, 1, False, True)
    step(2, 0, 2, True, True)
    step(3, 1, 3, True, True)

    def quad(r, carry):
        t0 = r * 4
        step(t0 + 0, 0, 0, True, True)
        step(t0 + 1, 1, 1, True, True)
        step(t0 + 2, 0, 2, True, True)
        step(t0 + 3, 1, 3, True, True)
        return carry

    lax.fori_loop(1, _NCH // 4 - 1, quad, 0)

    tl = _NCH - 4
    step(tl + 0, 0, 0, True, True)
    step(tl + 1, 1, 1, True, True)
    step(tl + 2, 0, 2, True, False)
    step(tl + 3, 1, 3, True, False)
    sdesc(0, 2).wait()
    sdesc(1, 3).wait()

    plsc.subcore_barrier()
    pltpu.sync_copy(acc.at[pl.ds(s * rps, rps)],
                    out.at[pl.ds(coff + s * rps, rps)])


def _seg_sum(xsplit, srcr, dstr, dh):
    zeros = jnp.zeros((_NPAD // _NSUB, dh), jnp.float32)
    mesh = plsc.VectorSubcoreMesh(core_axis_name="c", subcore_axis_name="s",
                                  num_cores=_NCORE, num_subcores=_NSUB)
    f = pl.kernel(
        functools.partial(_seg_sum_body, dh),
        out_type=jax.ShapeDtypeStruct((_NCORE * _NPAD, dh), jnp.float32),
        mesh=mesh,
        scratch_types=(
            [pltpu.VMEM((_K,), jnp.int32) for _ in range(8)]
            + [pltpu.VMEM((_K, dh), jnp.float32) for _ in range(2)]
            + [pltpu.VMEM_SHARED((_NPAD, dh), jnp.float32)]
            + [pltpu.SemaphoreType.DMA for _ in range(6)]
        ),
    )
    return f(xsplit, srcr, dstr, zeros)


def _bn_lrelu_split(z, g, bb, m, v, o_ref):
    scale = g * lax.rsqrt(v + 1e-5)
    z = (z - m) * scale + bb
    z = jnp.where(z >= 0, z, 0.01 * z)
    o_ref[0] = z[:, :128]
    o_ref[1] = z[:, 128:]


def _l1_body(x_ref, alo_ref, ahi_ref, ws_ref, wn_ref, b_ref, g_ref, bb_ref,
             m_ref, v_ref, o_ref):
    x = x_ref[...]
    a = jnp.concatenate([alo_ref[:, :64], ahi_ref[:, :64]], axis=1)
    deg = alo_ref[:, 64:65]
    hn = a * (1.0 / jnp.maximum(deg, 1.0))
    z = (jnp.dot(x, ws_ref[...], preferred_element_type=jnp.float32)
         + jnp.dot(hn, wn_ref[...], preferred_element_type=jnp.float32)
         + b_ref[...])
    _bn_lrelu_split(z, g_ref[...], bb_ref[...], m_ref[...], v_ref[...], o_ref)


def _l23_body(xlo_ref, xhi_ref, alo_ref, ahi_ref, d_ref, ws_ref, wn_ref,
              b_ref, g_ref, bb_ref, m_ref, v_ref, o_ref):
    x = jnp.concatenate([xlo_ref[...], xhi_ref[...]], axis=1)
    a = jnp.concatenate([alo_ref[...], ahi_ref[...]], axis=1)
    deg = d_ref[:, 64:65]
    hn = a * (1.0 / jnp.maximum(deg, 1.0))
    z = (jnp.dot(x, ws_ref[...], preferred_element_type=jnp.float32)
         + jnp.dot(hn, wn_ref[...], preferred_element_type=jnp.float32)
         + b_ref[...])
    _bn_lrelu_split(z, g_ref[...], bb_ref[...], m_ref[...], v_ref[...], o_ref)


def _full(shape):
    return pl.BlockSpec(shape, lambda i: (0,) * len(shape))


def _layer1(h_pad, agg1, Ws, Wn, b, g, bb, m, v):
    return pl.pallas_call(
        _l1_body,
        grid=(_NT,),
        in_specs=[
            pl.BlockSpec((_R, 128), lambda i: (i, 0)),
            pl.BlockSpec((_R, 128), lambda i: (i, 0)),
            pl.BlockSpec((_R, 128), lambda i: (_NT + i, 0)),
            _full((128, _H)), _full((128, _H)),
            _full((1, _H)), _full((1, _H)), _full((1, _H)),
            _full((1, _H)), _full((1, _H)),
        ],
        out_specs=pl.BlockSpec((2, _R, 128), lambda i: (0, i, 0)),
        out_shape=jax.ShapeDtypeStruct((2, _NPAD, 128), jnp.float32),
    )(h_pad, agg1, agg1, Ws, Wn, b, g, bb, m, v)


def _layer23(xsplit, agg, agg1, Ws, Wn, b, g, bb, m, v):
    return pl.pallas_call(
        _l23_body,
        grid=(_NT,),
        in_specs=[
            pl.BlockSpec((_R, 128), lambda i: (i, 0)),
            pl.BlockSpec((_R, 128), lambda i: (_NT + i, 0)),
            pl.BlockSpec((_R, 128), lambda i: (i, 0)),
            pl.BlockSpec((_R, 128), lambda i: (_NT + i, 0)),
            pl.BlockSpec((_R, 128), lambda i: (i, 0)),
            _full((_H, _H)), _full((_H, _H)),
            _full((1, _H)), _full((1, _H)), _full((1, _H)),
            _full((1, _H)), _full((1, _H)),
        ],
        out_specs=pl.BlockSpec((2, _R, 128), lambda i: (0, i, 0)),
        out_shape=jax.ShapeDtypeStruct((2, _NPAD, 128), jnp.float32),
    )(xsplit, xsplit, agg, agg, agg1, Ws, Wn, b, g, bb, m, v)


def _final_body(xlo_ref, xhi_ref, f1w_ref, f1b_ref, f2w_ref, f2b_ref,
                f3w_ref, f3b_ref, o_ref, acc_ref):
    i = pl.program_id(0)
    xt = jnp.concatenate([xlo_ref[...], xhi_ref[...]], axis=1)
    row = i * _R + lax.broadcasted_iota(jnp.int32, (_R, 1), 0)
    xt = jnp.where(row < _N, xt, 0.0)

    @pl.when(i == 0)
    def _():
        acc_ref[...] = jnp.zeros_like(acc_ref)

    acc_ref[...] += jnp.sum(xt, axis=0, keepdims=True)

    @pl.when(i == _NT - 1)
    def _():
        hg = acc_ref[...] * (1.0 / _N)
        y = hg @ f1w_ref[...] + f1b_ref[...]
        y = jnp.where(y >= 0, y, 0.01 * y)
        y = y @ f2w_ref[...] + f2b_ref[...]
        y = jnp.where(y >= 0, y, 0.01 * y)
        o_ref[...] = y @ f3w_ref[...] + f3b_ref[...]


def _final(xsplit, f1w, f1b, f2w, f2b, f3w, f3b):
    return pl.pallas_call(
        _final_body,
        grid=(_NT,),
        in_specs=[
            pl.BlockSpec((_R, 128), lambda i: (i, 0)),
            pl.BlockSpec((_R, 128), lambda i: (_NT + i, 0)),
            _full((_H, _H)), _full((1, _H)),
            _full((_H, 1024)), _full((1, 1024)),
            _full((1024, 128)), _full((1, 128)),
        ],
        out_specs=pl.BlockSpec((1, 128), lambda i: (0, 0)),
        out_shape=jax.ShapeDtypeStruct((1, 128), jnp.float32),
        scratch_shapes=[pltpu.VMEM((1, _H), jnp.float32)],
    )(xsplit, xsplit, f1w, f1b, f2w, f2b, f3w, f3b)


def kernel(h, edge_index, Ws1, Wn1, b1, Ws2, Wn2, b2, Ws3, Wn3, b3,
           bn1g, bn1b, bn1m, bn1v, bn2g, bn2b, bn2m, bn2v, bn3g, bn3b,
           bn3m, bn3v, fc1W, fc1b, fc2W, fc2b, fc3W, fc3b):
    f32 = jnp.float32
    npd = _EPAD - _E
    src = jnp.concatenate([edge_index[0], jnp.zeros((npd,), jnp.int32)])
    dst = jnp.concatenate([edge_index[1], jnp.full((npd,), _NPAD - 1, jnp.int32)])

    h_pad = jnp.zeros((_NPAD, 128), f32).at[:_N].set(h)
    onescol = jnp.ones((_NPAD, 1), f32)
    zpad = jnp.zeros((_NPAD, 63), f32)
    zpad2 = jnp.zeros((_NPAD, 64), f32)
    hsplit = jnp.concatenate([
        jnp.concatenate([h_pad[:, :64], onescol, zpad], axis=1),
        jnp.concatenate([h_pad[:, 64:], zpad2], axis=1)], axis=0)

    r1 = lambda a: a.reshape(1, -1)

    agg1 = _seg_sum(hsplit, src, dst, 128)
    x1 = _layer1(h_pad, agg1, Ws1, Wn1, r1(b1), r1(bn1g), r1(bn1b),
                 r1(bn1m), r1(bn1v)).reshape(_NCORE * _NPAD, 128)
    agg2 = _seg_sum(x1, src, dst, 128)
    x2 = _layer23(x1, agg2, agg1, Ws2, Wn2, r1(b2), r1(bn2g), r1(bn2b),
                  r1(bn2m), r1(bn2v)).reshape(_NCORE * _NPAD, 128)
    agg3 = _seg_sum(x2, src, dst, 128)
    x3 = _layer23(x2, agg3, agg1, Ws3, Wn3, r1(b3), r1(bn3g), r1(bn3b),
                  r1(bn3m), r1(bn3v)).reshape(_NCORE * _NPAD, 128)

    f3Wp = jnp.zeros((1024, 128), f32).at[:, :18].set(fc3W)
    f3bp = jnp.zeros((1, 128), f32).at[:, :18].set(r1(fc3b))
    y = _final(x3, fc1W, r1(fc1b), fc2W, r1(fc2b), f3Wp, f3bp)
    return y[:, :18]


# speedup vs baseline: 1.0071x; 1.0040x over previous
import functools

import jax
import jax.numpy as jnp
from jax import lax
from jax.experimental import pallas as pl
from jax.experimental.pallas import tpu as pltpu
from jax.experimental.pallas import tpu_sc as plsc

_N = 10000
_NPAD = 10240
_E = 320000
_R = 256
_NT = _NPAD // _R
_K = 128
_NSUB = 16
_NCORE = 2
_H = 256


_EPAD = 327680
_EPS = _EPAD // _NSUB
_NCH = _EPS // _K


def _seg_sum_body(dh, xsplit, srcr, dstr, zeros, out,
                  sv0, sv1, ix0, ix1, dv0, dv1, dv2, dv3, rw0, rw1, acc,
                  isem0, isem1, gsem0, gsem1):
    srcv = (sv0, sv1)
    idxv = (ix0, ix1)
    dstv = (dv0, dv1, dv2, dv3)
    rows = (rw0, rw1)
    isem = (isem0, isem1)
    gsem = (gsem0, gsem1)
    cax = lax.axis_index("c")
    s = lax.axis_index("s")
    rps = _NPAD // _NSUB
    pltpu.sync_copy(zeros, acc.at[pl.ds(s * rps, rps)])
    coff = cax * _NPAD
    ebase = s * _EPS

    def idescs(t, j, q):
        off = ebase + t * _K
        return (
            pltpu.make_async_copy(srcr.at[pl.ds(off, _K)], srcv[j], isem[j]),
            pltpu.make_async_copy(dstr.at[pl.ds(off, _K)], dstv[q], isem[j]),
        )

    def gdesc(j):
        return pltpu.make_async_copy(xsplit.at[idxv[j]], rows[j], gsem[j])

    def build(j):
        for g in range(_K // 16):
            idxv[j][pl.ds(g * 16, 16)] = srcv[j][pl.ds(g * 16, 16)] + coff

    def step(t, u, wait_next_idx, fire_next_idx, fire_next_gather):
        j = u % 2
        j1 = (u + 1) % 2
        if wait_next_idx:
            for d in idescs(t + 1, j1, (u + 1) % 4):
                d.wait()
            build(j1)
        gdesc(j).wait()
        if fire_next_gather:
            gdesc(j1).start()
        if fire_next_idx:
            for d in idescs(t + 2, j, (u + 2) % 4):
                d.start()
        pltpu.sync_copy(rows[j], acc.at[dstv[u]], add=True)

    for d in idescs(0, 0, 0):
        d.start()
    for d in idescs(1, 1, 1):
        d.start()
    plsc.subcore_barrier()
    for d in idescs(0, 0, 0):
        d.wait()
    build(0)
    gdesc(0).start()

    def quad(r, carry):
        t0 = r * 4
        for u in range(4):
            step(t0 + u, u, True, True, True)
        return carry

    lax.fori_loop(0, _NCH // 4 - 1, quad, 0)
    tl = _NCH - 4
    step(tl + 0, 0, True, True, True)
    step(tl + 1, 1, True, True, True)
    step(tl + 2, 2, True, False, True)
    step(tl + 3, 3, False, False, False)

    plsc.subcore_barrier()
    pltpu.sync_copy(acc.at[pl.ds(s * rps, rps)],
                    out.at[pl.ds(coff + s * rps, rps)])


def _seg_sum(xsplit, srcr, dstr, dh):
    zeros = jnp.zeros((_NPAD // _NSUB, dh), jnp.float32)
    mesh = plsc.VectorSubcoreMesh(core_axis_name="c", subcore_axis_name="s",
                                  num_cores=_NCORE, num_subcores=_NSUB)
    f = pl.kernel(
        functools.partial(_seg_sum_body, dh),
        out_type=jax.ShapeDtypeStruct((_NCORE * _NPAD, dh), jnp.float32),
        mesh=mesh,
        scratch_types=(
            [pltpu.VMEM((_K,), jnp.int32) for _ in range(8)]
            + [pltpu.VMEM((_K, dh), jnp.float32) for _ in range(2)]
            + [pltpu.VMEM_SHARED((_NPAD, dh), jnp.float32)]
            + [pltpu.SemaphoreType.DMA for _ in range(4)]
        ),
    )
    return f(xsplit, srcr, dstr, zeros)


def _bn_lrelu_split(z, g, bb, m, v, o_ref):
    scale = g * lax.rsqrt(v + 1e-5)
    z = (z - m) * scale + bb
    z = jnp.where(z >= 0, z, 0.01 * z)
    o_ref[0] = z[:, :128]
    o_ref[1] = z[:, 128:]


def _l1_body(x_ref, alo_ref, ahi_ref, ws_ref, wn_ref, b_ref, g_ref, bb_ref,
             m_ref, v_ref, o_ref):
    x = x_ref[...]
    a = jnp.concatenate([alo_ref[:, :64], ahi_ref[:, :64]], axis=1)
    deg = alo_ref[:, 64:65]
    hn = a * (1.0 / jnp.maximum(deg, 1.0))
    z = (jnp.dot(x, ws_ref[...], preferred_element_type=jnp.float32)
         + jnp.dot(hn, wn_ref[...], preferred_element_type=jnp.float32)
         + b_ref[...])
    _bn_lrelu_split(z, g_ref[...], bb_ref[...], m_ref[...], v_ref[...], o_ref)


def _l23_body(xlo_ref, xhi_ref, alo_ref, ahi_ref, d_ref, ws_ref, wn_ref,
              b_ref, g_ref, bb_ref, m_ref, v_ref, o_ref):
    x = jnp.concatenate([xlo_ref[...], xhi_ref[...]], axis=1)
    a = jnp.concatenate([alo_ref[...], ahi_ref[...]], axis=1)
    deg = d_ref[:, 64:65]
    hn = a * (1.0 / jnp.maximum(deg, 1.0))
    z = (jnp.dot(x, ws_ref[...], preferred_element_type=jnp.float32)
         + jnp.dot(hn, wn_ref[...], preferred_element_type=jnp.float32)
         + b_ref[...])
    _bn_lrelu_split(z, g_ref[...], bb_ref[...], m_ref[...], v_ref[...], o_ref)


def _full(shape):
    return pl.BlockSpec(shape, lambda i: (0,) * len(shape))


def _layer1(h_pad, agg1, Ws, Wn, b, g, bb, m, v):
    return pl.pallas_call(
        _l1_body,
        grid=(_NT,),
        in_specs=[
            pl.BlockSpec((_R, 128), lambda i: (i, 0)),
            pl.BlockSpec((_R, 128), lambda i: (i, 0)),
            pl.BlockSpec((_R, 128), lambda i: (_NT + i, 0)),
            _full((128, _H)), _full((128, _H)),
            _full((1, _H)), _full((1, _H)), _full((1, _H)),
            _full((1, _H)), _full((1, _H)),
        ],
        out_specs=pl.BlockSpec((2, _R, 128), lambda i: (0, i, 0)),
        out_shape=jax.ShapeDtypeStruct((2, _NPAD, 128), jnp.float32),
    )(h_pad, agg1, agg1, Ws, Wn, b, g, bb, m, v)


def _layer23(xsplit, agg, agg1, Ws, Wn, b, g, bb, m, v):
    return pl.pallas_call(
        _l23_body,
        grid=(_NT,),
        in_specs=[
            pl.BlockSpec((_R, 128), lambda i: (i, 0)),
            pl.BlockSpec((_R, 128), lambda i: (_NT + i, 0)),
            pl.BlockSpec((_R, 128), lambda i: (i, 0)),
            pl.BlockSpec((_R, 128), lambda i: (_NT + i, 0)),
            pl.BlockSpec((_R, 128), lambda i: (i, 0)),
            _full((_H, _H)), _full((_H, _H)),
            _full((1, _H)), _full((1, _H)), _full((1, _H)),
            _full((1, _H)), _full((1, _H)),
        ],
        out_specs=pl.BlockSpec((2, _R, 128), lambda i: (0, i, 0)),
        out_shape=jax.ShapeDtypeStruct((2, _NPAD, 128), jnp.float32),
    )(xsplit, xsplit, agg, agg, agg1, Ws, Wn, b, g, bb, m, v)


def _final_body(xlo_ref, xhi_ref, f1w_ref, f1b_ref, f2w_ref, f2b_ref,
                f3w_ref, f3b_ref, o_ref, acc_ref):
    i = pl.program_id(0)
    xt = jnp.concatenate([xlo_ref[...], xhi_ref[...]], axis=1)
    row = i * _R + lax.broadcasted_iota(jnp.int32, (_R, 1), 0)
    xt = jnp.where(row < _N, xt, 0.0)

    @pl.when(i == 0)
    def _():
        acc_ref[...] = jnp.zeros_like(acc_ref)

    acc_ref[...] += jnp.sum(xt, axis=0, keepdims=True)

    @pl.when(i == _NT - 1)
    def _():
        hg = acc_ref[...] * (1.0 / _N)
        y = hg @ f1w_ref[...] + f1b_ref[...]
        y = jnp.where(y >= 0, y, 0.01 * y)
        y = y @ f2w_ref[...] + f2b_ref[...]
        y = jnp.where(y >= 0, y, 0.01 * y)
        o_ref[...] = y @ f3w_ref[...] + f3b_ref[...]


def _final(xsplit, f1w, f1b, f2w, f2b, f3w, f3b):
    return pl.pallas_call(
        _final_body,
        grid=(_NT,),
        in_specs=[
            pl.BlockSpec((_R, 128), lambda i: (i, 0)),
            pl.BlockSpec((_R, 128), lambda i: (_NT + i, 0)),
            _full((_H, _H)), _full((1, _H)),
            _full((_H, 1024)), _full((1, 1024)),
            _full((1024, 128)), _full((1, 128)),
        ],
        out_specs=pl.BlockSpec((1, 128), lambda i: (0, 0)),
        out_shape=jax.ShapeDtypeStruct((1, 128), jnp.float32),
        scratch_shapes=[pltpu.VMEM((1, _H), jnp.float32)],
    )(xsplit, xsplit, f1w, f1b, f2w, f2b, f3w, f3b)


def kernel(h, edge_index, Ws1, Wn1, b1, Ws2, Wn2, b2, Ws3, Wn3, b3,
           bn1g, bn1b, bn1m, bn1v, bn2g, bn2b, bn2m, bn2v, bn3g, bn3b,
           bn3m, bn3v, fc1W, fc1b, fc2W, fc2b, fc3W, fc3b):
    f32 = jnp.float32
    npd = _EPAD - _E
    src = jnp.concatenate([edge_index[0], jnp.zeros((npd,), jnp.int32)])
    dst = jnp.concatenate([edge_index[1], jnp.full((npd,), _NPAD - 1, jnp.int32)])

    h_pad = jnp.zeros((_NPAD, 128), f32).at[:_N].set(h)
    onescol = jnp.ones((_NPAD, 1), f32)
    zpad = jnp.zeros((_NPAD, 63), f32)
    zpad2 = jnp.zeros((_NPAD, 64), f32)
    hsplit = jnp.concatenate([
        jnp.concatenate([h_pad[:, :64], onescol, zpad], axis=1),
        jnp.concatenate([h_pad[:, 64:], zpad2], axis=1)], axis=0)

    r1 = lambda a: a.reshape(1, -1)

    agg1 = _seg_sum(hsplit, src, dst, 128)
    x1 = _layer1(h_pad, agg1, Ws1, Wn1, r1(b1), r1(bn1g), r1(bn1b),
                 r1(bn1m), r1(bn1v)).reshape(_NCORE * _NPAD, 128)
    agg2 = _seg_sum(x1, src, dst, 128)
    x2 = _layer23(x1, agg2, agg1, Ws2, Wn2, r1(b2), r1(bn2g), r1(bn2b),
                  r1(bn2m), r1(bn2v)).reshape(_NCORE * _NPAD, 128)
    agg3 = _seg_sum(x2, src, dst, 128)
    x3 = _layer23(x2, agg3, agg1, Ws3, Wn3, r1(b3), r1(bn3g), r1(bn3b),
                  r1(bn3m), r1(bn3v)).reshape(_NCORE * _NPAD, 128)

    f3Wp = jnp.zeros((1024, 128), f32).at[:, :18].set(fc3W)
    f3bp = jnp.zeros((1, 128), f32).at[:, :18].set(r1(fc3b))
    y = _final(x3, fc1W, r1(fc1b), fc2W, r1(fc2b), f3Wp, f3bp)
    return y[:, :18]


# speedup vs baseline: 1.4769x; 1.4664x over previous
import functools

import jax
import jax.numpy as jnp
from jax import lax
from jax.experimental import pallas as pl
from jax.experimental.pallas import tpu as pltpu
from jax.experimental.pallas import tpu_sc as plsc

_N = 10000
_NPAD = 10240
_E = 320000
_R = 256
_NT = _NPAD // _R
_K = 128
_NSUB = 16
_NCORE = 2
_H = 256


_NCHT = _E // _K


def _seg_sum_body(dh, xsplit, combo, zeros, out, ebuf, rows, acc, gsem):
    cax = lax.axis_index("c")
    s = lax.axis_index("s")
    rps = _NPAD // _NSUB
    pltpu.sync_copy(zeros, acc.at[pl.ds(s * rps, rps)])
    plsc.subcore_barrier()
    cbase = cax * (2 * _NCHT)
    base_n = _NCHT // _NSUB
    rem = _NCHT - base_n * _NSUB
    nch = base_n + jnp.where(s < rem, 1, 0)

    def body(t, carry):
        ci = s + t * _NSUB
        pltpu.sync_copy(combo.at[pl.ds(cbase + ci * 2, 2)], ebuf)
        pltpu.async_copy(xsplit.at[ebuf.at[0]], rows, gsem).wait()
        pltpu.sync_copy(rows, acc.at[ebuf.at[1]], add=True)
        return carry

    lax.fori_loop(0, nch, body, 0)
    plsc.subcore_barrier()
    pltpu.sync_copy(acc.at[pl.ds(s * rps, rps)],
                    out.at[pl.ds(cax * _NPAD + s * rps, rps)])


def _seg_sum(xsplit, combo, dh):
    zeros = jnp.zeros((_NPAD // _NSUB, dh), jnp.float32)
    mesh = plsc.VectorSubcoreMesh(core_axis_name="c", subcore_axis_name="s",
                                  num_cores=_NCORE, num_subcores=_NSUB)
    f = pl.kernel(
        functools.partial(_seg_sum_body, dh),
        out_type=jax.ShapeDtypeStruct((_NCORE * _NPAD, dh), jnp.float32),
        mesh=mesh,
        scratch_types=[
            pltpu.VMEM((2, _K), jnp.int32),
            pltpu.VMEM((_K, dh), jnp.float32),
            pltpu.VMEM_SHARED((_NPAD, dh), jnp.float32),
            pltpu.SemaphoreType.DMA,
        ],
    )
    return f(xsplit, combo, zeros)


def _bn_lrelu_split(z, g, bb, m, v, o_ref):
    scale = g * lax.rsqrt(v + 1e-5)
    z = (z - m) * scale + bb
    z = jnp.where(z >= 0, z, 0.01 * z)
    o_ref[0] = z[:, :128]
    o_ref[1] = z[:, 128:]


def _l1_body(x_ref, alo_ref, ahi_ref, ws_ref, wn_ref, b_ref, g_ref, bb_ref,
             m_ref, v_ref, o_ref):
    x = x_ref[...]
    a = jnp.concatenate([alo_ref[:, :64], ahi_ref[:, :64]], axis=1)
    deg = alo_ref[:, 64:65]
    hn = a * (1.0 / jnp.maximum(deg, 1.0))
    z = (jnp.dot(x, ws_ref[...], preferred_element_type=jnp.float32)
         + jnp.dot(hn, wn_ref[...], preferred_element_type=jnp.float32)
         + b_ref[...])
    _bn_lrelu_split(z, g_ref[...], bb_ref[...], m_ref[...], v_ref[...], o_ref)


def _l23_body(xlo_ref, xhi_ref, alo_ref, ahi_ref, d_ref, ws_ref, wn_ref,
              b_ref, g_ref, bb_ref, m_ref, v_ref, o_ref):
    x = jnp.concatenate([xlo_ref[...], xhi_ref[...]], axis=1)
    a = jnp.concatenate([alo_ref[...], ahi_ref[...]], axis=1)
    deg = d_ref[:, 64:65]
    hn = a * (1.0 / jnp.maximum(deg, 1.0))
    z = (jnp.dot(x, ws_ref[...], preferred_element_type=jnp.float32)
         + jnp.dot(hn, wn_ref[...], preferred_element_type=jnp.float32)
         + b_ref[...])
    _bn_lrelu_split(z, g_ref[...], bb_ref[...], m_ref[...], v_ref[...], o_ref)


def _full(shape):
    return pl.BlockSpec(shape, lambda i: (0,) * len(shape))


def _layer1(h_pad, agg1, Ws, Wn, b, g, bb, m, v):
    return pl.pallas_call(
        _l1_body,
        grid=(_NT,),
        in_specs=[
            pl.BlockSpec((_R, 128), lambda i: (i, 0)),
            pl.BlockSpec((_R, 128), lambda i: (i, 0)),
            pl.BlockSpec((_R, 128), lambda i: (_NT + i, 0)),
            _full((128, _H)), _full((128, _H)),
            _full((1, _H)), _full((1, _H)), _full((1, _H)),
            _full((1, _H)), _full((1, _H)),
        ],
        out_specs=pl.BlockSpec((2, _R, 128), lambda i: (0, i, 0)),
        out_shape=jax.ShapeDtypeStruct((2, _NPAD, 128), jnp.float32),
    )(h_pad, agg1, agg1, Ws, Wn, b, g, bb, m, v)


def _layer23(xsplit, agg, agg1, Ws, Wn, b, g, bb, m, v):
    return pl.pallas_call(
        _l23_body,
        grid=(_NT,),
        in_specs=[
            pl.BlockSpec((_R, 128), lambda i: (i, 0)),
            pl.BlockSpec((_R, 128), lambda i: (_NT + i, 0)),
            pl.BlockSpec((_R, 128), lambda i: (i, 0)),
            pl.BlockSpec((_R, 128), lambda i: (_NT + i, 0)),
            pl.BlockSpec((_R, 128), lambda i: (i, 0)),
            _full((_H, _H)), _full((_H, _H)),
            _full((1, _H)), _full((1, _H)), _full((1, _H)),
            _full((1, _H)), _full((1, _H)),
        ],
        out_specs=pl.BlockSpec((2, _R, 128), lambda i: (0, i, 0)),
        out_shape=jax.ShapeDtypeStruct((2, _NPAD, 128), jnp.float32),
    )(xsplit, xsplit, agg, agg, agg1, Ws, Wn, b, g, bb, m, v)


def _final_body(xlo_ref, xhi_ref, f1w_ref, f1b_ref, f2w_ref, f2b_ref,
                f3w_ref, f3b_ref, o_ref, acc_ref):
    i = pl.program_id(0)
    xt = jnp.concatenate([xlo_ref[...], xhi_ref[...]], axis=1)
    row = i * _R + lax.broadcasted_iota(jnp.int32, (_R, 1), 0)
    xt = jnp.where(row < _N, xt, 0.0)

    @pl.when(i == 0)
    def _():
        acc_ref[...] = jnp.zeros_like(acc_ref)

    acc_ref[...] += jnp.sum(xt, axis=0, keepdims=True)

    @pl.when(i == _NT - 1)
    def _():
        hg = acc_ref[...] * (1.0 / _N)
        y = hg @ f1w_ref[...] + f1b_ref[...]
        y = jnp.where(y >= 0, y, 0.01 * y)
        y = y @ f2w_ref[...] + f2b_ref[...]
        y = jnp.where(y >= 0, y, 0.01 * y)
        o_ref[...] = y @ f3w_ref[...] + f3b_ref[...]


def _final(xsplit, f1w, f1b, f2w, f2b, f3w, f3b):
    return pl.pallas_call(
        _final_body,
        grid=(_NT,),
        in_specs=[
            pl.BlockSpec((_R, 128), lambda i: (i, 0)),
            pl.BlockSpec((_R, 128), lambda i: (_NT + i, 0)),
            _full((_H, _H)), _full((1, _H)),
            _full((_H, 1024)), _full((1, 1024)),
            _full((1024, 128)), _full((1, 128)),
        ],
        out_specs=pl.BlockSpec((1, 128), lambda i: (0, 0)),
        out_shape=jax.ShapeDtypeStruct((1, 128), jnp.float32),
        scratch_shapes=[pltpu.VMEM((1, _H), jnp.float32)],
    )(xsplit, xsplit, f1w, f1b, f2w, f2b, f3w, f3b)


def kernel(h, edge_index, Ws1, Wn1, b1, Ws2, Wn2, b2, Ws3, Wn3, b3,
           bn1g, bn1b, bn1m, bn1v, bn2g, bn2b, bn2m, bn2v, bn3g, bn3b,
           bn3m, bn3v, fc1W, fc1b, fc2W, fc2b, fc3W, fc3b):
    f32 = jnp.float32
    srcm = edge_index[0].reshape(_NCHT, 1, _K)
    dstm = edge_index[1].reshape(_NCHT, 1, _K)
    combo = jnp.concatenate(
        [jnp.concatenate([srcm + c * _NPAD, dstm], axis=1) for c in (0, 1)],
        axis=0).reshape(2 * _NCHT * 2, _K)

    h_pad = jnp.zeros((_NPAD, 128), f32).at[:_N].set(h)
    onescol = jnp.ones((_NPAD, 1), f32)
    zpad = jnp.zeros((_NPAD, 63), f32)
    zpad2 = jnp.zeros((_NPAD, 64), f32)
    hsplit = jnp.concatenate([
        jnp.concatenate([h_pad[:, :64], onescol, zpad], axis=1),
        jnp.concatenate([h_pad[:, 64:], zpad2], axis=1)], axis=0)

    r1 = lambda a: a.reshape(1, -1)

    agg1 = _seg_sum(hsplit, combo, 128)
    x1 = _layer1(h_pad, agg1, Ws1, Wn1, r1(b1), r1(bn1g), r1(bn1b),
                 r1(bn1m), r1(bn1v)).reshape(_NCORE * _NPAD, 128)
    agg2 = _seg_sum(x1, combo, 128)
    x2 = _layer23(x1, agg2, agg1, Ws2, Wn2, r1(b2), r1(bn2g), r1(bn2b),
                  r1(bn2m), r1(bn2v)).reshape(_NCORE * _NPAD, 128)
    agg3 = _seg_sum(x2, combo, 128)
    x3 = _layer23(x2, agg3, agg1, Ws3, Wn3, r1(b3), r1(bn3g), r1(bn3b),
                  r1(bn3m), r1(bn3v)).reshape(_NCORE * _NPAD, 128)

    f3Wp = jnp.zeros((1024, 128), f32).at[:, :18].set(fc3W)
    f3bp = jnp.zeros((1, 128), f32).at[:, :18].set(r1(fc3b))
    y = _final(x3, fc1W, r1(fc1b), fc2W, r1(fc2b), f3Wp, f3bp)
    return y[:, :18]
